# baseline (speedup 1.0000x reference)
import functools

import jax
import jax.numpy as jnp
from jax import lax
from jax.experimental import pallas as pl
from jax.experimental.pallas import tpu as pltpu
from jax.experimental.pallas import tpu_sc as plsc

N = 10000
E = 320000
D = 128
DE = 16
H = 128
LEAKY = 0.2
ROW = H + 16
NC = 2
NS = 16
NW = NC * NS
EPW = E // NW
CG = 400
NCG = EPW // CG
CA = 80
NCA = EPW // CA
NP = 10240
TPR = NP // NS

_HI = jax.lax.Precision.HIGHEST


def _leaky(v):
    return jnp.maximum(v, LEAKY * v)


def _dot_t(a, b):
    return lax.dot_general(a, b, (((1,), (1,)), ((), ())),
                           precision=_HI, preferred_element_type=jnp.float32)


def _node_body(x_ref, w1_ref, b1_ref, wa_ref, ba_ref, watt_ref, batt_ref,
               w2_ref, h_ref, p_ref, t_ref, s1r_ref):
    x = x_ref[...]
    h = _leaky(_dot_t(x, w1_ref[...]) + b1_ref[...])
    h_ref[...] = h
    t_ref[...] = _dot_t(h, wa_ref[...]) + ba_ref[...]
    p_ref[...] = _dot_t(x, w2_ref[:, :D]).astype(jnp.bfloat16)
    s1r_ref[...] = _dot_t(h, watt_ref[...]) + batt_ref[0, 0]


def _node_kernel(x, W1, b1r, Wa, bar, Watt, battr, W2):
    R = 1000
    g = N // R
    return pl.pallas_call(
        _node_body,
        grid=(g,),
        in_specs=[
            pl.BlockSpec((R, D), lambda i: (i, 0)),
            pl.BlockSpec((H, D), lambda i: (0, 0)),
            pl.BlockSpec((1, H), lambda i: (0, 0)),
            pl.BlockSpec((H, H), lambda i: (0, 0)),
            pl.BlockSpec((1, H), lambda i: (0, 0)),
            pl.BlockSpec((16, H), lambda i: (0, 0)),
            pl.BlockSpec((1, 1), lambda i: (0, 0)),
            pl.BlockSpec((H, D + DE), lambda i: (0, 0)),
        ],
        out_specs=[
            pl.BlockSpec((R, H), lambda i: (i, 0)),
            pl.BlockSpec((R, H), lambda i: (i, 0)),
            pl.BlockSpec((R, H), lambda i: (i, 0)),
            pl.BlockSpec((R, 16), lambda i: (i, 0)),
        ],
        out_shape=[
            jax.ShapeDtypeStruct((N, H), jnp.float32),
            jax.ShapeDtypeStruct((N, H), jnp.bfloat16),
            jax.ShapeDtypeStruct((N, H), jnp.float32),
            jax.ShapeDtypeStruct((N, 16), jnp.float32),
        ],
    )(x, W1, b1r, Wa, bar, Watt, battr, W2)


def _sc_gather_body(p_hbm, s1r_hbm, ei_hbm, psrc_hbm, s1d_hbm,
                    src_v, dst_v, p_buf, s1_buf):
    cid = lax.axis_index("c")
    sid = lax.axis_index("s")
    wid = sid * NC + cid
    base = wid * EPW

    def _chunk(ci, _):
        e0 = base + ci * CG
        pltpu.sync_copy(ei_hbm.at[0, pl.ds(e0, CG)], src_v)
        pltpu.sync_copy(ei_hbm.at[1, pl.ds(e0, CG)], dst_v)
        pltpu.sync_copy(p_hbm.at[src_v], p_buf)
        pltpu.sync_copy(s1r_hbm.at[dst_v], s1_buf)
        pltpu.sync_copy(p_buf, psrc_hbm.at[pl.ds(e0, CG)])
        pltpu.sync_copy(s1_buf, s1d_hbm.at[pl.ds(e0, CG)])
        return _
    lax.fori_loop(0, NCG, _chunk, None)


def _sc_gather_kernel(p, s1r, ei):
    mesh = plsc.VectorSubcoreMesh(core_axis_name="c", subcore_axis_name="s")
    f = functools.partial(
        pl.kernel, mesh=mesh,
        compiler_params=pltpu.CompilerParams(use_tc_tiling_on_sc=False,
                                             needs_layout_passes=False),
        out_type=[
            jax.ShapeDtypeStruct((E, H), jnp.bfloat16),
            jax.ShapeDtypeStruct((E, 16), jnp.float32),
        ],
        scratch_types=[
            pltpu.VMEM((CG,), jnp.int32),
            pltpu.VMEM((CG,), jnp.int32),
            pltpu.VMEM((CG, H), jnp.bfloat16),
            pltpu.VMEM((CG, 16), jnp.float32),
        ],
    )(_sc_gather_body)
    return f(p, s1r, ei)


def _score_body(psrc_ref, s1d_ref, ea_ref, w2_ref, b2_ref, wb_ref, w_ref):
    q = _dot_t(ea_ref[...], w2_ref[:, D:]) + b2_ref[...]
    nb = _leaky(psrc_ref[...].astype(jnp.float32) + q)
    s2 = _dot_t(nb, wb_ref[...])
    w_ref[...] = jnp.exp(_leaky(s1d_ref[...] + s2))


def _score_kernel(psrc, s1d, ea, W2, b2r, wb16):
    R = 4000
    g = E // R
    return pl.pallas_call(
        _score_body,
        grid=(g,),
        in_specs=[
            pl.BlockSpec((R, H), lambda i: (i, 0)),
            pl.BlockSpec((R, 16), lambda i: (i, 0)),
            pl.BlockSpec((R, DE), lambda i: (i, 0)),
            pl.BlockSpec((H, D + DE), lambda i: (0, 0)),
            pl.BlockSpec((1, H), lambda i: (0, 0)),
            pl.BlockSpec((16, H), lambda i: (0, 0)),
        ],
        out_specs=pl.BlockSpec((R, 16), lambda i: (i, 0)),
        out_shape=jax.ShapeDtypeStruct((E, 16), jnp.float32),
    )(psrc, s1d, ea, W2, b2r, wb16)


def _sc_agg_body(t_hbm, w_hbm, ei_hbm, out_hbm,
                 idx2_v, t_buf, w_buf, out_v, acc_sh):
    cid = lax.axis_index("c")
    sid = lax.axis_index("s")
    wid = sid * NC + cid

    def _zrow(i, _):
        for k in range(ROW // 16):
            out_v[i, pl.ds(k * 16, 16)] = jnp.zeros((16,), jnp.float32)
        return _
    lax.fori_loop(0, CA, _zrow, None)
    for k in range(TPR // CA):
        pltpu.sync_copy(out_v, acc_sh.at[pl.ds(sid * TPR + k * CA, CA)])
    plsc.subcore_barrier()

    base = wid * EPW

    def _chunk(ci, _):
        e0 = base + ci * CA
        pltpu.sync_copy(ei_hbm.at[:, pl.ds(e0, CA)], idx2_v)
        pltpu.sync_copy(t_hbm.at[idx2_v.at[0]], t_buf)
        pltpu.sync_copy(w_hbm.at[pl.ds(e0, CA)], w_buf)

        for e in range(CA):
            w16 = w_buf[e, pl.ds(0, 16)]
            for r in range(H // 16):
                out_v[e, pl.ds(r * 16, 16)] = t_buf[e, pl.ds(r * 16, 16)] * w16
            out_v[e, pl.ds(H, 16)] = w16

        pltpu.sync_copy(out_v, acc_sh.at[idx2_v.at[1]], add=True)
        return _
    lax.fori_loop(0, NCA, _chunk, None)

    plsc.subcore_barrier()
    for k in range(TPR // CA):
        r0 = sid * TPR + k * CA
        pltpu.sync_copy(acc_sh.at[pl.ds(r0, CA)], out_v)
        pltpu.sync_copy(out_v, out_hbm.at[cid, pl.ds(r0, CA)])


def _sc_agg_kernel(t, w, ei):
    mesh = plsc.VectorSubcoreMesh(core_axis_name="c", subcore_axis_name="s")
    f = functools.partial(
        pl.kernel, mesh=mesh,
        compiler_params=pltpu.CompilerParams(use_tc_tiling_on_sc=False,
                                             needs_layout_passes=False),
        out_type=jax.ShapeDtypeStruct((NC, NP, ROW), jnp.float32),
        scratch_types=[
            pltpu.VMEM((2, CA), jnp.int32),
            pltpu.VMEM((CA, H), jnp.float32),
            pltpu.VMEM((CA, 16), jnp.float32),
            pltpu.VMEM((CA, ROW), jnp.float32),
            pltpu.VMEM_SHARED((NP, ROW), jnp.float32),
        ],
    )(_sc_agg_body)
    return f(t, w, ei)


def _post_body(acc_ref, h_ref, wih_ref, whh_ref, bih_ref, bhh_ref, out_ref):
    a = acc_ref[...]
    s = a[0] + a[1]
    numer = s[:, :H]
    denom = s[:, H:H + 1]
    transform = numer / jnp.maximum(denom, 1e-16)
    context = jnp.where(transform > 0, transform,
                        jnp.exp(jnp.minimum(transform, 0.0)) - 1.0)
    h = h_ref[...]
    oe = jnp.where(denom > 0, context, h)
    gi = _dot_t(oe, wih_ref[...]) + bih_ref[...]
    gh = _dot_t(h, whh_ref[...]) + bhh_ref[...]
    r = jax.nn.sigmoid(gi[:, :H] + gh[:, :H])
    z = jax.nn.sigmoid(gi[:, H:2 * H] + gh[:, H:2 * H])
    n = jnp.tanh(gi[:, 2 * H:] + r * gh[:, 2 * H:])
    out_ref[...] = (1.0 - z) * n + z * h


def _post_kernel(acc, h, Wih, Whh, bihr, bhhr):
    R = 1000
    g = N // R
    return pl.pallas_call(
        _post_body,
        grid=(g,),
        in_specs=[
            pl.BlockSpec((NC, R, ROW), lambda i: (0, i, 0)),
            pl.BlockSpec((R, H), lambda i: (i, 0)),
            pl.BlockSpec((3 * H, H), lambda i: (0, 0)),
            pl.BlockSpec((3 * H, H), lambda i: (0, 0)),
            pl.BlockSpec((1, 3 * H), lambda i: (0, 0)),
            pl.BlockSpec((1, 3 * H), lambda i: (0, 0)),
        ],
        out_specs=pl.BlockSpec((R, H), lambda i: (i, 0)),
        out_shape=jax.ShapeDtypeStruct((N, H), jnp.float32),
    )(acc, h, Wih, Whh, bihr, bhhr)


def kernel(atom_features, edge_index, edge_attr, W1, b1, W2, b2, Watt, batt,
           Wa, ba, Wih, Whh, bih, bhh):
    b1r = b1.reshape(1, H)
    bar = ba.reshape(1, H)
    b2r = b2.reshape(1, H)
    battr = batt.reshape(1, 1)
    bihr = bih.reshape(1, 3 * H)
    bhhr = bhh.reshape(1, 3 * H)
    watt16 = jnp.broadcast_to(Watt[:, :H], (16, H))
    wb16 = jnp.broadcast_to(Watt[:, H:], (16, H))

    h, p, t, s1r = _node_kernel(atom_features, W1, b1r, Wa, bar, watt16, battr, W2)
    psrc, s1d = _sc_gather_kernel(p, s1r, edge_index)
    w = _score_kernel(psrc, s1d, edge_attr, W2, b2r, wb16)
    acc = _sc_agg_kernel(t, w, edge_index)
    new_h = _post_kernel(acc, h, Wih, Whh, bihr, bhhr)
    return (new_h, h)


# speedup vs baseline: 1.3296x; 1.3296x over previous
import functools

import jax
import jax.numpy as jnp
from jax import lax
from jax.experimental import pallas as pl
from jax.experimental.pallas import tpu as pltpu
from jax.experimental.pallas import tpu_sc as plsc

N = 10000
E = 320000
D = 128
DE = 16
H = 128
LEAKY = 0.2
ROW = H + 16
NC = 2
NS = 16
NW = NC * NS
EPW = E // NW
CG = 400
NCG = EPW // CG
CA = 80
NCA = EPW // CA
NP = 10240
TPR = NP // NS

_HI = jax.lax.Precision.HIGHEST


def _leaky(v):
    return jnp.maximum(v, LEAKY * v)


def _dot_t(a, b):
    return lax.dot_general(a, b, (((1,), (1,)), ((), ())),
                           precision=_HI, preferred_element_type=jnp.float32)


def _node_body(x_ref, w1_ref, b1_ref, wa_ref, ba_ref, watt_ref, batt_ref,
               w2_ref, h_ref, p_ref, t_ref, s1r_ref):
    x = x_ref[...]
    h = _leaky(_dot_t(x, w1_ref[...]) + b1_ref[...])
    h_ref[...] = h
    t_ref[...] = _dot_t(h, wa_ref[...]) + ba_ref[...]
    p_ref[...] = _dot_t(x, w2_ref[:, :D])
    s1r_ref[...] = _dot_t(h, watt_ref[...]) + batt_ref[0, 0]


def _node_kernel(x, W1, b1r, Wa, bar, Watt, battr, W2):
    R = 1000
    g = N // R
    return pl.pallas_call(
        _node_body,
        grid=(g,),
        in_specs=[
            pl.BlockSpec((R, D), lambda i: (i, 0)),
            pl.BlockSpec((H, D), lambda i: (0, 0)),
            pl.BlockSpec((1, H), lambda i: (0, 0)),
            pl.BlockSpec((H, H), lambda i: (0, 0)),
            pl.BlockSpec((1, H), lambda i: (0, 0)),
            pl.BlockSpec((16, H), lambda i: (0, 0)),
            pl.BlockSpec((1, 1), lambda i: (0, 0)),
            pl.BlockSpec((H, D + DE), lambda i: (0, 0)),
        ],
        out_specs=[
            pl.BlockSpec((R, H), lambda i: (i, 0)),
            pl.BlockSpec((R, H), lambda i: (i, 0)),
            pl.BlockSpec((R, H), lambda i: (i, 0)),
            pl.BlockSpec((R, 16), lambda i: (i, 0)),
        ],
        out_shape=[
            jax.ShapeDtypeStruct((N, H), jnp.float32),
            jax.ShapeDtypeStruct((N, H), jnp.float32),
            jax.ShapeDtypeStruct((N, H), jnp.float32),
            jax.ShapeDtypeStruct((N, 16), jnp.float32),
        ],
    )(x, W1, b1r, Wa, bar, Watt, battr, W2)


def _sc_gather_body(p_hbm, s1r_hbm, ei_hbm, psrc_hbm, s1d_hbm,
                    src_v, dst_v, p_buf, s1_buf):
    cid = lax.axis_index("c")
    sid = lax.axis_index("s")
    wid = sid * NC + cid
    base = wid * EPW

    def _chunk(ci, _):
        e0 = base + ci * CG
        pltpu.sync_copy(ei_hbm.at[0, pl.ds(e0, CG)], src_v)
        pltpu.sync_copy(ei_hbm.at[1, pl.ds(e0, CG)], dst_v)
        pltpu.sync_copy(p_hbm.at[src_v], p_buf)
        pltpu.sync_copy(s1r_hbm.at[dst_v], s1_buf)
        pltpu.sync_copy(p_buf, psrc_hbm.at[pl.ds(e0, CG)])
        pltpu.sync_copy(s1_buf, s1d_hbm.at[pl.ds(e0, CG)])
        return _
    lax.fori_loop(0, NCG, _chunk, None)


def _sc_gather_kernel(p, s1r, ei):
    mesh = plsc.VectorSubcoreMesh(core_axis_name="c", subcore_axis_name="s")
    f = functools.partial(
        pl.kernel, mesh=mesh,
        compiler_params=pltpu.CompilerParams(use_tc_tiling_on_sc=False,
                                             needs_layout_passes=False),
        out_type=[
            jax.ShapeDtypeStruct((E, H), jnp.float32),
            jax.ShapeDtypeStruct((E, 16), jnp.float32),
        ],
        scratch_types=[
            pltpu.VMEM((CG,), jnp.int32),
            pltpu.VMEM((CG,), jnp.int32),
            pltpu.VMEM((CG, H), jnp.float32),
            pltpu.VMEM((CG, 16), jnp.float32),
        ],
    )(_sc_gather_body)
    return f(p, s1r, ei)


def _score_body(psrc_ref, s1d_ref, ea_ref, w2_ref, b2_ref, wb_ref, w_ref):
    q = _dot_t(ea_ref[...], w2_ref[:, D:]) + b2_ref[...]
    nb = _leaky(psrc_ref[...] + q)
    s2 = _dot_t(nb, wb_ref[...])
    w_ref[...] = jnp.exp(_leaky(s1d_ref[...] + s2))


def _score_kernel(psrc, s1d, ea, W2, b2r, wb16):
    R = 4000
    g = E // R
    return pl.pallas_call(
        _score_body,
        grid=(g,),
        in_specs=[
            pl.BlockSpec((R, H), lambda i: (i, 0)),
            pl.BlockSpec((R, 16), lambda i: (i, 0)),
            pl.BlockSpec((R, DE), lambda i: (i, 0)),
            pl.BlockSpec((H, D + DE), lambda i: (0, 0)),
            pl.BlockSpec((1, H), lambda i: (0, 0)),
            pl.BlockSpec((16, H), lambda i: (0, 0)),
        ],
        out_specs=pl.BlockSpec((R, 16), lambda i: (i, 0)),
        out_shape=jax.ShapeDtypeStruct((E, 16), jnp.float32),
    )(psrc, s1d, ea, W2, b2r, wb16)


def _sc_agg_body(t_hbm, w_hbm, ei_hbm, out_hbm,
                 idx2_0, idx2_1, t_0, t_1, w_0, w_1, sem0, sem1,
                 out_v, acc_sh):
    cid = lax.axis_index("c")
    sid = lax.axis_index("s")
    wid = sid * NC + cid
    idx2 = (idx2_0, idx2_1)
    tb = (t_0, t_1)
    wb_ = (w_0, w_1)
    sems = (sem0, sem1)

    def _zrow(i, _):
        for k in range(ROW // 16):
            out_v[i, pl.ds(k * 16, 16)] = jnp.zeros((16,), jnp.float32)
        return _
    lax.fori_loop(0, CA, _zrow, None)
    for k in range(TPR // CA):
        pltpu.sync_copy(out_v, acc_sh.at[pl.ds(sid * TPR + k * CA, CA)])
    plsc.subcore_barrier()

    base = wid * EPW

    def _fetch(ci, b):
        e0 = base + ci * CA
        pltpu.sync_copy(ei_hbm.at[:, pl.ds(e0, CA)], idx2[b])
        pltpu.async_copy(t_hbm.at[idx2[b].at[0]], tb[b], sems[b])
        pltpu.async_copy(w_hbm.at[pl.ds(e0, CA)], wb_[b], sems[b])

    def _consume(ci, b):
        e0 = base + ci * CA
        pltpu.make_async_copy(t_hbm.at[idx2[b].at[0]], tb[b], sems[b]).wait()
        pltpu.make_async_copy(w_hbm.at[pl.ds(e0, CA)], wb_[b], sems[b]).wait()
        for e in range(CA):
            w16 = wb_[b][e, pl.ds(0, 16)]
            for r in range(H // 16):
                out_v[e, pl.ds(r * 16, 16)] = tb[b][e, pl.ds(r * 16, 16)] * w16
            out_v[e, pl.ds(H, 16)] = w16
        pltpu.sync_copy(out_v, acc_sh.at[idx2[b].at[1]], add=True)

    _fetch(0, 0)

    def _pair(i, _):
        for b in range(2):
            ci = 2 * i + b
            _fetch(ci + 1, b ^ 1)
            _consume(ci, b)
        return _
    lax.fori_loop(0, (NCA - 1) // 2, _pair, None)
    _consume(NCA - 1, (NCA - 1) % 2)

    plsc.subcore_barrier()
    for k in range(TPR // CA):
        r0 = sid * TPR + k * CA
        pltpu.sync_copy(acc_sh.at[pl.ds(r0, CA)], out_v)
        pltpu.sync_copy(out_v, out_hbm.at[cid, pl.ds(r0, CA)])


def _sc_agg_kernel(t, w, ei):
    mesh = plsc.VectorSubcoreMesh(core_axis_name="c", subcore_axis_name="s")
    f = functools.partial(
        pl.kernel, mesh=mesh,
        compiler_params=pltpu.CompilerParams(use_tc_tiling_on_sc=False,
                                             needs_layout_passes=False),
        out_type=jax.ShapeDtypeStruct((NC, NP, ROW), jnp.float32),
        scratch_types=[
            pltpu.VMEM((2, CA), jnp.int32),
            pltpu.VMEM((2, CA), jnp.int32),
            pltpu.VMEM((CA, H), jnp.float32),
            pltpu.VMEM((CA, H), jnp.float32),
            pltpu.VMEM((CA, 16), jnp.float32),
            pltpu.VMEM((CA, 16), jnp.float32),
            pltpu.SemaphoreType.DMA,
            pltpu.SemaphoreType.DMA,
            pltpu.VMEM((CA, ROW), jnp.float32),
            pltpu.VMEM_SHARED((NP, ROW), jnp.float32),
        ],
    )(_sc_agg_body)
    return f(t, w, ei)


def _post_body(acc_ref, h_ref, wih_ref, whh_ref, bih_ref, bhh_ref, out_ref):
    a = acc_ref[...]
    s = a[0] + a[1]
    numer = s[:, :H]
    denom = s[:, H:H + 1]
    transform = numer / jnp.maximum(denom, 1e-16)
    context = jnp.where(transform > 0, transform,
                        jnp.exp(jnp.minimum(transform, 0.0)) - 1.0)
    h = h_ref[...]
    oe = jnp.where(denom > 0, context, h)
    gi = _dot_t(oe, wih_ref[...]) + bih_ref[...]
    gh = _dot_t(h, whh_ref[...]) + bhh_ref[...]
    r = jax.nn.sigmoid(gi[:, :H] + gh[:, :H])
    z = jax.nn.sigmoid(gi[:, H:2 * H] + gh[:, H:2 * H])
    n = jnp.tanh(gi[:, 2 * H:] + r * gh[:, 2 * H:])
    out_ref[...] = (1.0 - z) * n + z * h


def _post_kernel(acc, h, Wih, Whh, bihr, bhhr):
    R = 1000
    g = N // R
    return pl.pallas_call(
        _post_body,
        grid=(g,),
        in_specs=[
            pl.BlockSpec((NC, R, ROW), lambda i: (0, i, 0)),
            pl.BlockSpec((R, H), lambda i: (i, 0)),
            pl.BlockSpec((3 * H, H), lambda i: (0, 0)),
            pl.BlockSpec((3 * H, H), lambda i: (0, 0)),
            pl.BlockSpec((1, 3 * H), lambda i: (0, 0)),
            pl.BlockSpec((1, 3 * H), lambda i: (0, 0)),
        ],
        out_specs=pl.BlockSpec((R, H), lambda i: (i, 0)),
        out_shape=jax.ShapeDtypeStruct((N, H), jnp.float32),
    )(acc, h, Wih, Whh, bihr, bhhr)


def kernel(atom_features, edge_index, edge_attr, W1, b1, W2, b2, Watt, batt,
           Wa, ba, Wih, Whh, bih, bhh):
    b1r = b1.reshape(1, H)
    bar = ba.reshape(1, H)
    b2r = b2.reshape(1, H)
    battr = batt.reshape(1, 1)
    bihr = bih.reshape(1, 3 * H)
    bhhr = bhh.reshape(1, 3 * H)
    watt16 = jnp.broadcast_to(Watt[:, :H], (16, H))
    wb16 = jnp.broadcast_to(Watt[:, H:], (16, H))

    h, p, t, s1r = _node_kernel(atom_features, W1, b1r, Wa, bar, watt16, battr, W2)
    psrc, s1d = _sc_gather_kernel(p, s1r, edge_index)
    w = _score_kernel(psrc, s1d, edge_attr, W2, b2r, wb16)
    acc = _sc_agg_kernel(t, w, edge_index)
    new_h = _post_kernel(acc, h, Wih, Whh, bihr, bhhr)
    return (new_h, h)


# speedup vs baseline: 1.4047x; 1.0564x over previous
import functools

import jax
import jax.numpy as jnp
from jax import lax
from jax.experimental import pallas as pl
from jax.experimental.pallas import tpu as pltpu
from jax.experimental.pallas import tpu_sc as plsc

N = 10000
E = 320000
D = 128
DE = 16
H = 128
LEAKY = 0.2
ROW = H + 16
NC = 2
NS = 16
NW = NC * NS
EPW = E // NW
CG = 400
NCG = EPW // CG
CA = 40
NCA = EPW // CA
NP = 10240
TPR = NP // NS

_HI = jax.lax.Precision.HIGHEST


def _leaky(v):
    return jnp.maximum(v, LEAKY * v)


def _dot_t(a, b):
    return lax.dot_general(a, b, (((1,), (1,)), ((), ())),
                           precision=_HI, preferred_element_type=jnp.float32)


def _node_body(x_ref, w1_ref, b1_ref, wa_ref, ba_ref, watt_ref, batt_ref,
               w2_ref, h_ref, p_ref, t_ref, s1r_ref):
    x = x_ref[...]
    h = _leaky(_dot_t(x, w1_ref[...]) + b1_ref[...])
    h_ref[...] = h
    t_ref[...] = _dot_t(h, wa_ref[...]) + ba_ref[...]
    p_ref[...] = _dot_t(x, w2_ref[:, :D])
    s1r_ref[...] = _dot_t(h, watt_ref[...]) + batt_ref[0, 0]


def _node_kernel(x, W1, b1r, Wa, bar, Watt, battr, W2):
    R = 1000
    g = N // R
    return pl.pallas_call(
        _node_body,
        grid=(g,),
        in_specs=[
            pl.BlockSpec((R, D), lambda i: (i, 0)),
            pl.BlockSpec((H, D), lambda i: (0, 0)),
            pl.BlockSpec((1, H), lambda i: (0, 0)),
            pl.BlockSpec((H, H), lambda i: (0, 0)),
            pl.BlockSpec((1, H), lambda i: (0, 0)),
            pl.BlockSpec((H, H), lambda i: (0, 0)),
            pl.BlockSpec((1, 1), lambda i: (0, 0)),
            pl.BlockSpec((H, D + DE), lambda i: (0, 0)),
        ],
        out_specs=[
            pl.BlockSpec((R, H), lambda i: (i, 0)),
            pl.BlockSpec((R, H), lambda i: (i, 0)),
            pl.BlockSpec((R, H), lambda i: (i, 0)),
            pl.BlockSpec((R, H), lambda i: (i, 0)),
        ],
        out_shape=[
            jax.ShapeDtypeStruct((N, H), jnp.float32),
            jax.ShapeDtypeStruct((N, H), jnp.float32),
            jax.ShapeDtypeStruct((N, H), jnp.float32),
            jax.ShapeDtypeStruct((N, H), jnp.float32),
        ],
    )(x, W1, b1r, Wa, bar, Watt, battr, W2)


def _sc_gather_body(p_hbm, s1r_hbm, ei_hbm, psrc_hbm, s1d_hbm,
                    src_v, dst_v, p_buf, s1_buf):
    cid = lax.axis_index("c")
    sid = lax.axis_index("s")
    wid = sid * NC + cid
    base = wid * EPW

    def _chunk(ci, _):
        e0 = base + ci * CG
        pltpu.sync_copy(ei_hbm.at[0, pl.ds(e0, CG)], src_v)
        pltpu.sync_copy(ei_hbm.at[1, pl.ds(e0, CG)], dst_v)
        pltpu.sync_copy(p_hbm.at[src_v], p_buf)
        pltpu.sync_copy(s1r_hbm.at[dst_v], s1_buf)
        pltpu.sync_copy(p_buf, psrc_hbm.at[pl.ds(e0, CG)])
        pltpu.sync_copy(s1_buf, s1d_hbm.at[pl.ds(e0, CG)])
        return _
    lax.fori_loop(0, NCG, _chunk, None)


def _sc_gather_kernel(p, s1r, ei):
    mesh = plsc.VectorSubcoreMesh(core_axis_name="c", subcore_axis_name="s")
    f = functools.partial(
        pl.kernel, mesh=mesh,
        compiler_params=pltpu.CompilerParams(use_tc_tiling_on_sc=False,
                                             needs_layout_passes=False),
        out_type=[
            jax.ShapeDtypeStruct((E, H), jnp.float32),
            jax.ShapeDtypeStruct((E, H), jnp.float32),
        ],
        scratch_types=[
            pltpu.VMEM((CG,), jnp.int32),
            pltpu.VMEM((CG,), jnp.int32),
            pltpu.VMEM((CG, H), jnp.float32),
            pltpu.VMEM((CG, H), jnp.float32),
        ],
    )(_sc_gather_body)
    return f(p, s1r, ei)


def _score_body(psrc_ref, s1d_ref, ea_ref, w2_ref, b2_ref, wb_ref, w_ref):
    q = _dot_t(ea_ref[...], w2_ref[:, D:]) + b2_ref[...]
    nb = _leaky(psrc_ref[...] + q)
    s2 = _dot_t(nb, wb_ref[...])
    w_ref[...] = jnp.exp(_leaky(s1d_ref[...] + s2))


def _score_kernel(psrc, s1d, ea, W2, b2r, wb128):
    R = 4000
    g = E // R
    return pl.pallas_call(
        _score_body,
        grid=(g,),
        in_specs=[
            pl.BlockSpec((R, H), lambda i: (i, 0)),
            pl.BlockSpec((R, H), lambda i: (i, 0)),
            pl.BlockSpec((R, DE), lambda i: (i, 0)),
            pl.BlockSpec((H, D + DE), lambda i: (0, 0)),
            pl.BlockSpec((1, H), lambda i: (0, 0)),
            pl.BlockSpec((H, H), lambda i: (0, 0)),
        ],
        out_specs=pl.BlockSpec((R, H), lambda i: (i, 0)),
        out_shape=jax.ShapeDtypeStruct((E, H), jnp.float32),
    )(psrc, s1d, ea, W2, b2r, wb128)


def _sc_agg_body(t_hbm, w_hbm, ei_hbm, out_hbm,
                 idx2_0, idx2_1, t_0, t_1, w_0, w_1, sem0, sem1,
                 out_v, acc_sh):
    cid = lax.axis_index("c")
    sid = lax.axis_index("s")
    wid = sid * NC + cid
    idx2 = (idx2_0, idx2_1)
    tb = (t_0, t_1)
    wb_ = (w_0, w_1)
    sems = (sem0, sem1)

    def _zrow(i, _):
        for k in range(ROW // 16):
            out_v[i, pl.ds(k * 16, 16)] = jnp.zeros((16,), jnp.float32)
        return _
    lax.fori_loop(0, CA, _zrow, None)
    for k in range(TPR // CA):
        pltpu.sync_copy(out_v, acc_sh.at[pl.ds(sid * TPR + k * CA, CA)])
    plsc.subcore_barrier()

    base = wid * EPW

    def _fetch(ci, b):
        e0 = base + ci * CA
        pltpu.sync_copy(ei_hbm.at[:, pl.ds(e0, CA)], idx2[b])
        pltpu.async_copy(t_hbm.at[idx2[b].at[0]], tb[b], sems[b])
        pltpu.async_copy(w_hbm.at[pl.ds(e0, CA)], wb_[b], sems[b])

    def _consume(ci, b):
        e0 = base + ci * CA
        pltpu.make_async_copy(t_hbm.at[idx2[b].at[0]], tb[b], sems[b]).wait()
        pltpu.make_async_copy(w_hbm.at[pl.ds(e0, CA)], wb_[b], sems[b]).wait()
        for e in range(CA):
            w16 = wb_[b][e, pl.ds(0, 16)]
            for r in range(H // 16):
                out_v[e, pl.ds(r * 16, 16)] = tb[b][e, pl.ds(r * 16, 16)] * w16
            out_v[e, pl.ds(H, 16)] = w16
        pltpu.sync_copy(out_v, acc_sh.at[idx2[b].at[1]], add=True)

    _fetch(0, 0)

    def _pair(i, _):
        for b in range(2):
            ci = 2 * i + b
            _fetch(ci + 1, b ^ 1)
            _consume(ci, b)
        return _
    lax.fori_loop(0, (NCA - 2) // 2, _pair, None)
    _fetch(NCA - 1, 1)
    _consume(NCA - 2, 0)
    _consume(NCA - 1, 1)

    plsc.subcore_barrier()
    for k in range(TPR // CA):
        r0 = sid * TPR + k * CA
        pltpu.sync_copy(acc_sh.at[pl.ds(r0, CA)], out_v)
        pltpu.sync_copy(out_v, out_hbm.at[cid, pl.ds(r0, CA)])


def _sc_agg_kernel(t, w, ei):
    mesh = plsc.VectorSubcoreMesh(core_axis_name="c", subcore_axis_name="s")
    f = functools.partial(
        pl.kernel, mesh=mesh,
        compiler_params=pltpu.CompilerParams(use_tc_tiling_on_sc=False,
                                             needs_layout_passes=False),
        out_type=jax.ShapeDtypeStruct((NC, NP, ROW), jnp.float32),
        scratch_types=[
            pltpu.VMEM((2, CA), jnp.int32),
            pltpu.VMEM((2, CA), jnp.int32),
            pltpu.VMEM((CA, H), jnp.float32),
            pltpu.VMEM((CA, H), jnp.float32),
            pltpu.VMEM((CA, H), jnp.float32),
            pltpu.VMEM((CA, H), jnp.float32),
            pltpu.SemaphoreType.DMA,
            pltpu.SemaphoreType.DMA,
            pltpu.VMEM((CA, ROW), jnp.float32),
            pltpu.VMEM_SHARED((NP, ROW), jnp.float32),
        ],
    )(_sc_agg_body)
    return f(t, w, ei)


def _post_body(acc_ref, h_ref, wih_ref, whh_ref, bih_ref, bhh_ref, out_ref):
    a = acc_ref[...]
    s = a[0] + a[1]
    numer = s[:, :H]
    denom = s[:, H:H + 1]
    transform = numer / jnp.maximum(denom, 1e-16)
    context = jnp.where(transform > 0, transform,
                        jnp.exp(jnp.minimum(transform, 0.0)) - 1.0)
    h = h_ref[...]
    oe = jnp.where(denom > 0, context, h)
    gi = _dot_t(oe, wih_ref[...]) + bih_ref[...]
    gh = _dot_t(h, whh_ref[...]) + bhh_ref[...]
    r = jax.nn.sigmoid(gi[:, :H] + gh[:, :H])
    z = jax.nn.sigmoid(gi[:, H:2 * H] + gh[:, H:2 * H])
    n = jnp.tanh(gi[:, 2 * H:] + r * gh[:, 2 * H:])
    out_ref[...] = (1.0 - z) * n + z * h


def _post_kernel(acc, h, Wih, Whh, bihr, bhhr):
    R = 1000
    g = N // R
    return pl.pallas_call(
        _post_body,
        grid=(g,),
        in_specs=[
            pl.BlockSpec((NC, R, ROW), lambda i: (0, i, 0)),
            pl.BlockSpec((R, H), lambda i: (i, 0)),
            pl.BlockSpec((3 * H, H), lambda i: (0, 0)),
            pl.BlockSpec((3 * H, H), lambda i: (0, 0)),
            pl.BlockSpec((1, 3 * H), lambda i: (0, 0)),
            pl.BlockSpec((1, 3 * H), lambda i: (0, 0)),
        ],
        out_specs=pl.BlockSpec((R, H), lambda i: (i, 0)),
        out_shape=jax.ShapeDtypeStruct((N, H), jnp.float32),
    )(acc, h, Wih, Whh, bihr, bhhr)


def kernel(atom_features, edge_index, edge_attr, W1, b1, W2, b2, Watt, batt,
           Wa, ba, Wih, Whh, bih, bhh):
    b1r = b1.reshape(1, H)
    bar = ba.reshape(1, H)
    b2r = b2.reshape(1, H)
    battr = batt.reshape(1, 1)
    bihr = bih.reshape(1, 3 * H)
    bhhr = bhh.reshape(1, 3 * H)
    watt128 = jnp.broadcast_to(Watt[:, :H], (H, H))
    wb128 = jnp.broadcast_to(Watt[:, H:], (H, H))

    h, p, t, s1r = _node_kernel(atom_features, W1, b1r, Wa, bar, watt128, battr, W2)
    psrc, s1d = _sc_gather_kernel(p, s1r, edge_index)
    w = _score_kernel(psrc, s1d, edge_attr, W2, b2r, wb128)
    acc = _sc_agg_kernel(t, w, edge_index)
    new_h = _post_kernel(acc, h, Wih, Whh, bihr, bhhr)
    return (new_h, h)
